# Initial kernel scaffold; baseline (speedup 1.0000x reference)
import jax
import jax.numpy as jnp
from jax.experimental import pallas as pl


def kernel(priors_xywha, gt_boxes, gt_labels):
    raise NotImplementedError("write your pallas kernel here")


# speedup vs baseline: 3.5203x; 3.5203x over previous
import functools

import jax
import jax.numpy as jnp
from jax import lax
from jax.experimental import pallas as pl
from jax.experimental.pallas import tpu as pltpu
from jax.experimental.pallas import tpu_sc as plsc

B, T, N = 8, 64, 20000
CHUNK = 5008
NJ = CHUNK // 16
LAST_BASE = N - CHUNK
LN2 = 0.6931472
SQRT2 = 1.4142135


def _log(x):
    bits = plsc.bitcast(x, jnp.int32)
    e = (bits >> 23) - 127
    mbits = (bits & 0x007FFFFF) | 0x3F800000
    m = plsc.bitcast(mbits, jnp.float32)
    big = m > SQRT2
    m = jnp.where(big, m * 0.5, m)
    e = jnp.where(big, e + 1, e)
    s = (m - 1.0) / (m + 1.0)
    s2 = s * s
    p = s * (2.0 + s2 * (0.66666667 + s2 * (0.4 + s2 * 0.2857143)))
    return e.astype(jnp.float32) * LN2 + p


def _body(pcx_hbm, pcy_hbm, pw_hbm, ph_hbm, bx1_hbm, by1_hbm, bx2_hbm, by2_hbm,
          labels_hbm, loc_hbm, lab_hbm,
          p_cx, p_cy, p_w, p_h, p_x1, p_y1, p_x2, p_y2, p_area,
          bx1, by1, bx2, by2, barea, blab,
          mval, midx, tbv, tbj, tred_v, tred_g, mg_v, mg_g, bp,
          loc_out, lab_out, sh_v, sh_g):
    cid = lax.axis_index("c")
    sid = lax.axis_index("s")
    b = cid * 4 + sid // 4
    g = sid % 4
    base = jnp.where(g == 3, LAST_BASE, g * CHUNK)
    iota = lax.iota(jnp.int32, 16)

    pltpu.sync_copy(pcx_hbm.at[pl.ds(base, CHUNK)], p_cx)
    pltpu.sync_copy(pcy_hbm.at[pl.ds(base, CHUNK)], p_cy)
    pltpu.sync_copy(pw_hbm.at[pl.ds(base, CHUNK)], p_w)
    pltpu.sync_copy(ph_hbm.at[pl.ds(base, CHUNK)], p_h)
    bsl = pl.ds(b * T, T)
    pltpu.sync_copy(bx1_hbm.at[bsl], bx1.at[pl.ds(0, T)])
    pltpu.sync_copy(by1_hbm.at[bsl], by1.at[pl.ds(0, T)])
    pltpu.sync_copy(bx2_hbm.at[bsl], bx2.at[pl.ds(0, T)])
    pltpu.sync_copy(by2_hbm.at[bsl], by2.at[pl.ds(0, T)])
    pltpu.sync_copy(labels_hbm.at[bsl], blab)

    def derive(j, _):
        sl = pl.ds(j * 16, 16)
        cx, cy, w, h = p_cx[sl], p_cy[sl], p_w[sl], p_h[sl]
        x1 = cx - w / 2.0
        y1 = cy - h / 2.0
        x2 = cx + w / 2.0
        y2 = cy + h / 2.0
        p_x1[sl] = x1
        p_y1[sl] = y1
        p_x2[sl] = x2
        p_y2[sl] = y2
        p_area[sl] = (x2 - x1) * (y2 - y1)
        return 0

    lax.fori_loop(0, NJ, derive, 0)

    def tinit(k, _):
        sl = pl.ds(k * 16, 16)
        x1, y1, x2, y2 = bx1[sl], by1[sl], bx2[sl], by2[sl]
        barea[sl] = (x2 - x1) * (y2 - y1)
        return 0

    lax.fori_loop(0, 4, tinit, 0)

    neg1 = jnp.full((16,), -1.0, jnp.float32)
    zero_i = jnp.full((16,), 0, jnp.int32)

    def tbinit(k, _):
        sl = pl.ds(k * 16, 16)
        tbv[sl] = neg1
        tbj[sl] = zero_i
        return 0

    lax.fori_loop(0, T, tbinit, 0)

    def minit(j, _):
        sl = pl.ds(j * 16, 16)
        mval[sl] = neg1
        midx[sl] = zero_i
        return 0

    lax.fori_loop(0, NJ, minit, 0)

    def tloop(t, _):
        tsl = pl.ds(t, 16)
        a_x1 = jnp.full((16,), bx1[tsl][0], jnp.float32)
        a_y1 = jnp.full((16,), by1[tsl][0], jnp.float32)
        a_x2 = jnp.full((16,), bx2[tsl][0], jnp.float32)
        a_y2 = jnp.full((16,), by2[tsl][0], jnp.float32)
        a_ar = jnp.full((16,), barea[tsl][0], jnp.float32)
        tvec = jnp.full((16,), t, jnp.int32)

        def jloop(j, carry):
            tv, tj = carry
            sl = pl.ds(j * 16, 16)
            px1, py1, px2, py2, pa = p_x1[sl], p_y1[sl], p_x2[sl], p_y2[sl], p_area[sl]
            wx = jnp.minimum(px2, a_x2) - jnp.maximum(px1, a_x1)
            wy = jnp.minimum(py2, a_y2) - jnp.maximum(py1, a_y1)
            inter = jnp.maximum(wx, 0.0) * jnp.maximum(wy, 0.0)
            denom = ((a_ar + pa) - inter) + 1e-12
            iou = inter / denom
            bv = mval[sl]
            c1 = iou > bv
            mval[sl] = jnp.where(c1, iou, bv)
            midx[sl] = jnp.where(c1, tvec, midx[sl])
            c2 = iou > tv
            tv = jnp.where(c2, iou, tv)
            tj = jnp.where(c2, jnp.full((16,), j, jnp.int32), tj)
            return tv, tj

        tv, tj = lax.fori_loop(0, NJ, jloop, (neg1, zero_i))
        osl = pl.ds(t * 16, 16)
        tbv[osl] = tv
        tbj[osl] = tj
        return 0

    lax.fori_loop(0, T, tloop, 0)

    for tgrp in range(4):
        cur_v = neg1
        cur_g = zero_i
        tvec = jnp.full((16,), tgrp * 16, jnp.int32) + iota
        for l in range(16):
            idx = tvec * 16 + l
            v_l = plsc.load_gather(tbv, [idx])
            j_l = plsc.load_gather(tbj, [idx])
            g_l = base + (j_l * 16 + l)
            take = (v_l > cur_v) | ((v_l == cur_v) & (g_l < cur_g))
            cur_v = jnp.where(take, v_l, cur_v)
            cur_g = jnp.where(take, g_l, cur_g)
        osl = pl.ds(tgrp * 16, 16)
        tred_v[osl] = cur_v
        tred_g[osl] = cur_g

    pltpu.sync_copy(tred_v, sh_v.at[pl.ds(sid * T, T)])
    pltpu.sync_copy(tred_g, sh_g.at[pl.ds(sid * T, T)])
    plsc.subcore_barrier()
    grp0 = (sid // 4) * 4
    pltpu.sync_copy(sh_v.at[pl.ds(grp0 * T, 4 * T)], mg_v)
    pltpu.sync_copy(sh_g.at[pl.ds(grp0 * T, 4 * T)], mg_g)
    for tt in range(4):
        cur_v = neg1
        cur_g = zero_i
        for gg in range(4):
            sl = pl.ds(gg * T + tt * 16, 16)
            v = mg_v[sl]
            gi = mg_g[sl]
            take = (v > cur_v) | ((v == cur_v) & (gi < cur_g))
            cur_v = jnp.where(take, v, cur_v)
            cur_g = jnp.where(take, gi, cur_g)
        bp[pl.ds(tt * 16, 16)] = cur_g

    lane0 = iota == 0

    def force(t, _):
        lp = bp[pl.ds(t, 16)][0] - base

        @pl.when((lp >= 0) & (lp < CHUNK))
        def _():
            li = jnp.full((16,), lp, jnp.int32)
            plsc.store_scatter(midx, [li], jnp.full((16,), t, jnp.int32), mask=lane0)
            plsc.store_scatter(mval, [li], jnp.full((16,), 2.0, jnp.float32), mask=lane0)

        return 0

    lax.fori_loop(0, T, force, 0)

    def encode(j, _):
        sl = pl.ds(j * 16, 16)
        m = midx[sl]
        v = mval[sl]
        lab = plsc.load_gather(blab, [m])
        lab_out[sl] = jnp.where(v < 0.5, jnp.full((16,), 0, jnp.int32), lab)
        m_x1 = plsc.load_gather(bx1, [m])
        m_y1 = plsc.load_gather(by1, [m])
        m_x2 = plsc.load_gather(bx2, [m])
        m_y2 = plsc.load_gather(by2, [m])
        cx, cy, w, h = p_cx[sl], p_cy[sl], p_w[sl], p_h[sl]
        g_cx = ((m_x1 + m_x2) / 2.0 - cx) / (0.1 * w)
        g_cy = ((m_y1 + m_y2) / 2.0 - cy) / (0.1 * h)
        g_w = _log(jnp.maximum((m_x2 - m_x1) / w, 1e-8)) / 0.2
        g_h = _log(jnp.maximum((m_y2 - m_y1) / h, 1e-8)) / 0.2
        rows4 = iota * 4 + jnp.full((16,), j * 64, jnp.int32)
        plsc.store_scatter(loc_out, [rows4], g_cx)
        plsc.store_scatter(loc_out, [rows4 + 1], g_cy)
        plsc.store_scatter(loc_out, [rows4 + 2], g_w)
        plsc.store_scatter(loc_out, [rows4 + 3], g_h)
        return 0

    lax.fori_loop(0, NJ, encode, 0)

    pltpu.sync_copy(loc_out, loc_hbm.at[pl.ds((b * N + base) * 4, CHUNK * 4)])
    pltpu.sync_copy(lab_out, lab_hbm.at[pl.ds(b * N + base, CHUNK)])


@jax.jit
def kernel(priors_xywha, gt_boxes, gt_labels):
    pcx, pcy, pw, ph = [jnp.reshape(priors_xywha[:, i], (N,)) for i in range(4)]
    b1, b2, b3, b4 = [jnp.reshape(gt_boxes[:, :, i], (B * T,)) for i in range(4)]
    labels = jnp.reshape(gt_labels.astype(jnp.int32), (B * T,))

    k = functools.partial(
        pl.kernel,
        out_type=(
            jax.ShapeDtypeStruct((B * N * 4,), jnp.float32),
            jax.ShapeDtypeStruct((B * N,), jnp.int32),
        ),
        mesh=plsc.VectorSubcoreMesh(core_axis_name="c", subcore_axis_name="s"),
        compiler_params=pltpu.CompilerParams(needs_layout_passes=False),
        scratch_types=(
            [pltpu.VMEM((CHUNK,), jnp.float32) for _ in range(9)]
            + [pltpu.VMEM((T + 16,), jnp.float32) for _ in range(5)]
            + [pltpu.VMEM((T,), jnp.int32)]
            + [pltpu.VMEM((CHUNK,), jnp.float32),
               pltpu.VMEM((CHUNK,), jnp.int32),
               pltpu.VMEM((T * 16,), jnp.float32),
               pltpu.VMEM((T * 16,), jnp.int32),
               pltpu.VMEM((T,), jnp.float32),
               pltpu.VMEM((T,), jnp.int32),
               pltpu.VMEM((4 * T,), jnp.float32),
               pltpu.VMEM((4 * T,), jnp.int32),
               pltpu.VMEM((T + 16,), jnp.int32),
               pltpu.VMEM((CHUNK * 4,), jnp.float32),
               pltpu.VMEM((CHUNK,), jnp.int32),
               pltpu.VMEM_SHARED((16 * T,), jnp.float32),
               pltpu.VMEM_SHARED((16 * T,), jnp.int32)]
        ),
    )(_body)
    loc_flat, lab_flat = k(pcx, pcy, pw, ph, b1, b2, b3, b4, labels)
    return loc_flat.reshape(B, N, 4), lab_flat.reshape(B, N)


# speedup vs baseline: 7.7271x; 2.1950x over previous
import functools

import jax
import jax.numpy as jnp
from jax import lax
from jax.experimental import pallas as pl
from jax.experimental.pallas import tpu as pltpu
from jax.experimental.pallas import tpu_sc as plsc

B, T, N = 8, 64, 20000
CHUNK = 5120
NJ = CHUNK // 16
GSTRIDE = 4960
LN2 = 0.6931472
SQRT2 = 1.4142135


def _log(x):
    bits = plsc.bitcast(x, jnp.int32)
    e = (bits >> 23) - 127
    mbits = (bits & 0x007FFFFF) | 0x3F800000
    m = plsc.bitcast(mbits, jnp.float32)
    big = m > SQRT2
    m = jnp.where(big, m * 0.5, m)
    e = jnp.where(big, e + 1, e)
    s = (m - 1.0) / (m + 1.0)
    s2 = s * s
    p = s * (2.0 + s2 * (0.66666667 + s2 * (0.4 + s2 * 0.2857143)))
    return e.astype(jnp.float32) * LN2 + p


def _body(pcx_hbm, pcy_hbm, pw_hbm, ph_hbm, bx1_hbm, by1_hbm, bx2_hbm, by2_hbm,
          labels_hbm, loc_hbm, lab_hbm,
          p_cx, p_cy, p_w, p_h, p_x1, p_y1, p_x2, p_y2, p_area,
          bx1, by1, bx2, by2, barea, blab,
          mval, midx, tbv, tbj, tred_v, tred_g, mg_v, mg_g, bp,
          loc_out, lab_out, sh_v, sh_g):
    cid = lax.axis_index("c")
    sid = lax.axis_index("s")
    b = cid * 4 + sid // 4
    g = sid % 4
    base = g * GSTRIDE
    iota = lax.iota(jnp.int32, 16)

    pltpu.sync_copy(pcx_hbm.at[pl.ds(base, CHUNK)], p_cx)
    pltpu.sync_copy(pcy_hbm.at[pl.ds(base, CHUNK)], p_cy)
    pltpu.sync_copy(pw_hbm.at[pl.ds(base, CHUNK)], p_w)
    pltpu.sync_copy(ph_hbm.at[pl.ds(base, CHUNK)], p_h)
    bsl = pl.ds(b * T, T)
    pltpu.sync_copy(bx1_hbm.at[bsl], bx1.at[pl.ds(0, T)])
    pltpu.sync_copy(by1_hbm.at[bsl], by1.at[pl.ds(0, T)])
    pltpu.sync_copy(bx2_hbm.at[bsl], bx2.at[pl.ds(0, T)])
    pltpu.sync_copy(by2_hbm.at[bsl], by2.at[pl.ds(0, T)])
    pltpu.sync_copy(labels_hbm.at[bsl], blab)

    @plsc.parallel_loop(0, NJ, unroll=4)
    def _derive(j):
        sl = pl.ds(j * 16, 16)
        cx, cy, w, h = p_cx[sl], p_cy[sl], p_w[sl], p_h[sl]
        x1 = cx - w / 2.0
        y1 = cy - h / 2.0
        x2 = cx + w / 2.0
        y2 = cy + h / 2.0
        p_x1[sl] = x1
        p_y1[sl] = y1
        p_x2[sl] = x2
        p_y2[sl] = y2
        p_area[sl] = (x2 - x1) * (y2 - y1)

    def tinit(k, _):
        sl = pl.ds(k * 16, 16)
        x1, y1, x2, y2 = bx1[sl], by1[sl], bx2[sl], by2[sl]
        barea[sl] = (x2 - x1) * (y2 - y1)
        return 0

    lax.fori_loop(0, 4, tinit, 0)

    neg1 = jnp.full((16,), -1.0, jnp.float32)
    zero_i = jnp.full((16,), 0, jnp.int32)

    def tbinit(k, _):
        sl = pl.ds(k * 16, 16)
        tbv[sl] = neg1
        tbj[sl] = zero_i
        return 0

    lax.fori_loop(0, T, tbinit, 0)

    @plsc.parallel_loop(0, NJ, unroll=4)
    def _minit(j):
        sl = pl.ds(j * 16, 16)
        mval[sl] = neg1
        midx[sl] = zero_i

    def tloop(t, _):
        tsl = pl.ds(t, 16)
        a_x1 = jnp.full((16,), bx1[tsl][0], jnp.float32)
        a_y1 = jnp.full((16,), by1[tsl][0], jnp.float32)
        a_x2 = jnp.full((16,), bx2[tsl][0], jnp.float32)
        a_y2 = jnp.full((16,), by2[tsl][0], jnp.float32)
        a_ar = jnp.full((16,), barea[tsl][0], jnp.float32)
        tvec = jnp.full((16,), t, jnp.int32)

        @plsc.parallel_loop(0, NJ, unroll=8, carry=(neg1, zero_i))
        def jloop(j, carry):
            tv, tg = carry
            sl = pl.ds(j * 16, 16)
            px1, py1, px2, py2, pa = p_x1[sl], p_y1[sl], p_x2[sl], p_y2[sl], p_area[sl]
            wx = jnp.minimum(px2, a_x2) - jnp.maximum(px1, a_x1)
            wy = jnp.minimum(py2, a_y2) - jnp.maximum(py1, a_y1)
            inter = jnp.maximum(wx, 0.0) * jnp.maximum(wy, 0.0)
            denom = ((a_ar + pa) - inter) + 1e-12
            iou = inter / denom
            bv = mval[sl]
            c1 = iou > bv
            mval[sl] = jnp.where(c1, iou, bv)
            midx[sl] = jnp.where(c1, tvec, midx[sl])
            gv = jnp.full((16,), base + j * 16, jnp.int32) + iota
            take = (iou > tv) | ((iou == tv) & (gv < tg))
            tv = jnp.where(take, iou, tv)
            tg = jnp.where(take, gv, tg)
            return tv, tg

        tv, tg = jloop
        osl = pl.ds(t * 16, 16)
        tbv[osl] = tv
        tbj[osl] = tg
        return 0

    lax.fori_loop(0, T, tloop, 0)

    for tgrp in range(4):
        cur_v = neg1
        cur_g = zero_i
        tvec = jnp.full((16,), tgrp * 16, jnp.int32) + iota
        for l in range(16):
            idx = tvec * 16 + l
            v_l = plsc.load_gather(tbv, [idx])
            g_l = plsc.load_gather(tbj, [idx])
            take = (v_l > cur_v) | ((v_l == cur_v) & (g_l < cur_g))
            cur_v = jnp.where(take, v_l, cur_v)
            cur_g = jnp.where(take, g_l, cur_g)
        osl = pl.ds(tgrp * 16, 16)
        tred_v[osl] = cur_v
        tred_g[osl] = cur_g

    pltpu.sync_copy(tred_v, sh_v.at[pl.ds(sid * T, T)])
    pltpu.sync_copy(tred_g, sh_g.at[pl.ds(sid * T, T)])
    plsc.subcore_barrier()
    grp0 = (sid // 4) * 4
    pltpu.sync_copy(sh_v.at[pl.ds(grp0 * T, 4 * T)], mg_v)
    pltpu.sync_copy(sh_g.at[pl.ds(grp0 * T, 4 * T)], mg_g)
    for tt in range(4):
        cur_v = neg1
        cur_g = zero_i
        for gg in range(4):
            sl = pl.ds(gg * T + tt * 16, 16)
            v = mg_v[sl]
            gi = mg_g[sl]
            take = (v > cur_v) | ((v == cur_v) & (gi < cur_g))
            cur_v = jnp.where(take, v, cur_v)
            cur_g = jnp.where(take, gi, cur_g)
        bp[pl.ds(tt * 16, 16)] = cur_g

    lane0 = iota == 0

    def force(t, _):
        lp = bp[pl.ds(t, 16)][0] - base

        @pl.when((lp >= 0) & (lp < CHUNK))
        def _():
            li = jnp.full((16,), lp, jnp.int32)
            plsc.store_scatter(midx, [li], jnp.full((16,), t, jnp.int32), mask=lane0)
            plsc.store_scatter(mval, [li], jnp.full((16,), 2.0, jnp.float32), mask=lane0)

        return 0

    lax.fori_loop(0, T, force, 0)

    @plsc.parallel_loop(0, NJ, unroll=4)
    def _encode(j):
        sl = pl.ds(j * 16, 16)
        m = midx[sl]
        v = mval[sl]
        lab = plsc.load_gather(blab, [m])
        lab_out[sl] = jnp.where(v < 0.5, jnp.full((16,), 0, jnp.int32), lab)
        m_x1 = plsc.load_gather(bx1, [m])
        m_y1 = plsc.load_gather(by1, [m])
        m_x2 = plsc.load_gather(bx2, [m])
        m_y2 = plsc.load_gather(by2, [m])
        cx, cy, w, h = p_cx[sl], p_cy[sl], p_w[sl], p_h[sl]
        g_cx = ((m_x1 + m_x2) / 2.0 - cx) / (0.1 * w)
        g_cy = ((m_y1 + m_y2) / 2.0 - cy) / (0.1 * h)
        g_w = _log(jnp.maximum((m_x2 - m_x1) / w, 1e-8)) / 0.2
        g_h = _log(jnp.maximum((m_y2 - m_y1) / h, 1e-8)) / 0.2
        rows4 = iota * 4 + jnp.full((16,), j * 64, jnp.int32)
        plsc.store_scatter(loc_out, [rows4], g_cx)
        plsc.store_scatter(loc_out, [rows4 + 1], g_cy)
        plsc.store_scatter(loc_out, [rows4 + 2], g_w)
        plsc.store_scatter(loc_out, [rows4 + 3], g_h)

    pltpu.sync_copy(loc_out, loc_hbm.at[pl.ds((b * N + base) * 4, CHUNK * 4)])
    pltpu.sync_copy(lab_out, lab_hbm.at[pl.ds(b * N + base, CHUNK)])


@jax.jit
def kernel(priors_xywha, gt_boxes, gt_labels):
    pcx, pcy, pw, ph = [jnp.reshape(priors_xywha[:, i], (N,)) for i in range(4)]
    b1, b2, b3, b4 = [jnp.reshape(gt_boxes[:, :, i], (B * T,)) for i in range(4)]
    labels = jnp.reshape(gt_labels.astype(jnp.int32), (B * T,))

    k = functools.partial(
        pl.kernel,
        out_type=(
            jax.ShapeDtypeStruct((B * N * 4,), jnp.float32),
            jax.ShapeDtypeStruct((B * N,), jnp.int32),
        ),
        mesh=plsc.VectorSubcoreMesh(core_axis_name="c", subcore_axis_name="s"),
        compiler_params=pltpu.CompilerParams(needs_layout_passes=False),
        scratch_types=(
            [pltpu.VMEM((CHUNK,), jnp.float32) for _ in range(9)]
            + [pltpu.VMEM((T + 16,), jnp.float32) for _ in range(5)]
            + [pltpu.VMEM((T,), jnp.int32)]
            + [pltpu.VMEM((CHUNK,), jnp.float32),
               pltpu.VMEM((CHUNK,), jnp.int32),
               pltpu.VMEM((T * 16,), jnp.float32),
               pltpu.VMEM((T * 16,), jnp.int32),
               pltpu.VMEM((T,), jnp.float32),
               pltpu.VMEM((T,), jnp.int32),
               pltpu.VMEM((4 * T,), jnp.float32),
               pltpu.VMEM((4 * T,), jnp.int32),
               pltpu.VMEM((T + 16,), jnp.int32),
               pltpu.VMEM((CHUNK * 4,), jnp.float32),
               pltpu.VMEM((CHUNK,), jnp.int32),
               pltpu.VMEM_SHARED((16 * T,), jnp.float32),
               pltpu.VMEM_SHARED((16 * T,), jnp.int32)]
        ),
    )(_body)
    loc_flat, lab_flat = k(pcx, pcy, pw, ph, b1, b2, b3, b4, labels)
    return loc_flat.reshape(B, N, 4), lab_flat.reshape(B, N)


# speedup vs baseline: 13.9546x; 1.8059x over previous
import functools

import jax
import jax.numpy as jnp
from jax import lax
from jax.experimental import pallas as pl
from jax.experimental.pallas import tpu as pltpu
from jax.experimental.pallas import tpu_sc as plsc

B, T, N = 8, 64, 20000
CHUNK = 5120
NJ = CHUNK // 16
GSTRIDE = 4960
LN2 = 0.6931472
SQRT2 = 1.4142135


def _log(x):
    bits = plsc.bitcast(x, jnp.int32)
    e = (bits >> 23) - 127
    mbits = (bits & 0x007FFFFF) | 0x3F800000
    m = plsc.bitcast(mbits, jnp.float32)
    big = m > SQRT2
    m = jnp.where(big, m * 0.5, m)
    e = jnp.where(big, e + 1, e)
    s = (m - 1.0) / (m + 1.0)
    s2 = s * s
    p = s * (2.0 + s2 * (0.66666667 + s2 * (0.4 + s2 * 0.2857143)))
    return e.astype(jnp.float32) * LN2 + p


def _body(pcx_hbm, pcy_hbm, pw_hbm, ph_hbm, bx1_hbm, by1_hbm, bx2_hbm, by2_hbm,
          labels_hbm, loc_hbm, lab_hbm,
          p_cx, p_cy, p_w, p_h, p_x1, p_y1, p_x2, p_y2, p_area,
          bx1, by1, bx2, by2, barea, blab,
          mval, midx, tbv, tbj, tred_v, tred_g, mg_v, mg_g, bp,
          lgx, lgy, lgw, lgh, lab_out, sh_v, sh_g):
    cid = lax.axis_index("c")
    sid = lax.axis_index("s")
    b = cid * 4 + sid // 4
    g = sid % 4
    base = g * GSTRIDE
    iota = lax.iota(jnp.int32, 16)

    pltpu.sync_copy(pcx_hbm.at[pl.ds(base, CHUNK)], p_cx)
    pltpu.sync_copy(pcy_hbm.at[pl.ds(base, CHUNK)], p_cy)
    pltpu.sync_copy(pw_hbm.at[pl.ds(base, CHUNK)], p_w)
    pltpu.sync_copy(ph_hbm.at[pl.ds(base, CHUNK)], p_h)
    bsl = pl.ds(b * T, T)
    pltpu.sync_copy(bx1_hbm.at[bsl], bx1.at[pl.ds(0, T)])
    pltpu.sync_copy(by1_hbm.at[bsl], by1.at[pl.ds(0, T)])
    pltpu.sync_copy(bx2_hbm.at[bsl], bx2.at[pl.ds(0, T)])
    pltpu.sync_copy(by2_hbm.at[bsl], by2.at[pl.ds(0, T)])
    pltpu.sync_copy(labels_hbm.at[bsl], blab)

    @plsc.parallel_loop(0, NJ, unroll=4)
    def _derive(j):
        sl = pl.ds(j * 16, 16)
        cx, cy, w, h = p_cx[sl], p_cy[sl], p_w[sl], p_h[sl]
        x1 = cx - w / 2.0
        y1 = cy - h / 2.0
        x2 = cx + w / 2.0
        y2 = cy + h / 2.0
        p_x1[sl] = x1
        p_y1[sl] = y1
        p_x2[sl] = x2
        p_y2[sl] = y2
        p_area[sl] = (x2 - x1) * (y2 - y1)

    def tinit(k, _):
        sl = pl.ds(k * 16, 16)
        x1, y1, x2, y2 = bx1[sl], by1[sl], bx2[sl], by2[sl]
        barea[sl] = (x2 - x1) * (y2 - y1)
        return 0

    lax.fori_loop(0, 4, tinit, 0)

    neg1 = jnp.full((16,), -1.0, jnp.float32)
    zero_i = jnp.full((16,), 0, jnp.int32)

    def tbinit(k, _):
        sl = pl.ds(k * 16, 16)
        tbv[sl] = neg1
        tbj[sl] = zero_i
        return 0

    lax.fori_loop(0, T, tbinit, 0)

    @plsc.parallel_loop(0, NJ, unroll=4)
    def _minit(j):
        sl = pl.ds(j * 16, 16)
        mval[sl] = neg1
        midx[sl] = zero_i

    def tloop(t, _):
        tsl = pl.ds(t, 16)
        a_x1 = jnp.full((16,), bx1[tsl][0], jnp.float32)
        a_y1 = jnp.full((16,), by1[tsl][0], jnp.float32)
        a_x2 = jnp.full((16,), bx2[tsl][0], jnp.float32)
        a_y2 = jnp.full((16,), by2[tsl][0], jnp.float32)
        a_ar = jnp.full((16,), barea[tsl][0], jnp.float32)
        tvec = jnp.full((16,), t, jnp.int32)

        @plsc.parallel_loop(0, NJ, unroll=8, carry=(neg1, zero_i))
        def jloop(j, carry):
            tv, tg = carry
            sl = pl.ds(j * 16, 16)
            px1, py1, px2, py2, pa = p_x1[sl], p_y1[sl], p_x2[sl], p_y2[sl], p_area[sl]
            wx = jnp.minimum(px2, a_x2) - jnp.maximum(px1, a_x1)
            wy = jnp.minimum(py2, a_y2) - jnp.maximum(py1, a_y1)
            inter = jnp.maximum(wx, 0.0) * jnp.maximum(wy, 0.0)
            denom = ((a_ar + pa) - inter) + 1e-12
            iou = inter / denom
            bv = mval[sl]
            c1 = iou > bv
            mval[sl] = jnp.where(c1, iou, bv)
            midx[sl] = jnp.where(c1, tvec, midx[sl])
            gv = jnp.full((16,), base + j * 16, jnp.int32) + iota
            take = (iou > tv) | ((iou == tv) & (gv < tg))
            tv = jnp.where(take, iou, tv)
            tg = jnp.where(take, gv, tg)
            return tv, tg

        tv, tg = jloop
        osl = pl.ds(t * 16, 16)
        tbv[osl] = tv
        tbj[osl] = tg
        return 0

    lax.fori_loop(0, T, tloop, 0)

    for tgrp in range(4):
        cur_v = neg1
        cur_g = zero_i
        tvec = jnp.full((16,), tgrp * 16, jnp.int32) + iota
        for l in range(16):
            idx = tvec * 16 + l
            v_l = plsc.load_gather(tbv, [idx])
            g_l = plsc.load_gather(tbj, [idx])
            take = (v_l > cur_v) | ((v_l == cur_v) & (g_l < cur_g))
            cur_v = jnp.where(take, v_l, cur_v)
            cur_g = jnp.where(take, g_l, cur_g)
        osl = pl.ds(tgrp * 16, 16)
        tred_v[osl] = cur_v
        tred_g[osl] = cur_g

    pltpu.sync_copy(tred_v, sh_v.at[pl.ds(sid * T, T)])
    pltpu.sync_copy(tred_g, sh_g.at[pl.ds(sid * T, T)])
    plsc.subcore_barrier()
    grp0 = (sid // 4) * 4
    pltpu.sync_copy(sh_v.at[pl.ds(grp0 * T, 4 * T)], mg_v)
    pltpu.sync_copy(sh_g.at[pl.ds(grp0 * T, 4 * T)], mg_g)
    for tt in range(4):
        cur_v = neg1
        cur_g = zero_i
        for gg in range(4):
            sl = pl.ds(gg * T + tt * 16, 16)
            v = mg_v[sl]
            gi = mg_g[sl]
            take = (v > cur_v) | ((v == cur_v) & (gi < cur_g))
            cur_v = jnp.where(take, v, cur_v)
            cur_g = jnp.where(take, gi, cur_g)
        bp[pl.ds(tt * 16, 16)] = cur_g

    lane0 = iota == 0

    def force(t, _):
        lp = bp[pl.ds(t, 16)][0] - base

        @pl.when((lp >= 0) & (lp < CHUNK))
        def _():
            li = jnp.full((16,), lp, jnp.int32)
            plsc.store_scatter(midx, [li], jnp.full((16,), t, jnp.int32), mask=lane0)
            plsc.store_scatter(mval, [li], jnp.full((16,), 2.0, jnp.float32), mask=lane0)

        return 0

    lax.fori_loop(0, T, force, 0)

    @plsc.parallel_loop(0, NJ, unroll=4)
    def _encode(j):
        sl = pl.ds(j * 16, 16)
        m = midx[sl]
        v = mval[sl]
        lab = plsc.load_gather(blab, [m])
        lab_out[sl] = jnp.where(v < 0.5, jnp.full((16,), 0, jnp.int32), lab)
        m_x1 = plsc.load_gather(bx1, [m])
        m_y1 = plsc.load_gather(by1, [m])
        m_x2 = plsc.load_gather(bx2, [m])
        m_y2 = plsc.load_gather(by2, [m])
        cx, cy, w, h = p_cx[sl], p_cy[sl], p_w[sl], p_h[sl]
        g_cx = ((m_x1 + m_x2) / 2.0 - cx) / (0.1 * w)
        g_cy = ((m_y1 + m_y2) / 2.0 - cy) / (0.1 * h)
        g_w = _log(jnp.maximum((m_x2 - m_x1) / w, 1e-8)) / 0.2
        g_h = _log(jnp.maximum((m_y2 - m_y1) / h, 1e-8)) / 0.2
        lgx[sl] = g_cx
        lgy[sl] = g_cy
        lgw[sl] = g_w
        lgh[sl] = g_h

    pltpu.sync_copy(lgx, loc_hbm.at[pl.ds((b * 4 + 0) * N + base, CHUNK)])
    pltpu.sync_copy(lgy, loc_hbm.at[pl.ds((b * 4 + 1) * N + base, CHUNK)])
    pltpu.sync_copy(lgw, loc_hbm.at[pl.ds((b * 4 + 2) * N + base, CHUNK)])
    pltpu.sync_copy(lgh, loc_hbm.at[pl.ds((b * 4 + 3) * N + base, CHUNK)])
    pltpu.sync_copy(lab_out, lab_hbm.at[pl.ds(b * N + base, CHUNK)])


@jax.jit
def kernel(priors_xywha, gt_boxes, gt_labels):
    pcx, pcy, pw, ph = [jnp.reshape(priors_xywha[:, i], (N,)) for i in range(4)]
    b1, b2, b3, b4 = [jnp.reshape(gt_boxes[:, :, i], (B * T,)) for i in range(4)]
    labels = jnp.reshape(gt_labels.astype(jnp.int32), (B * T,))

    k = functools.partial(
        pl.kernel,
        out_type=(
            jax.ShapeDtypeStruct((B * N * 4,), jnp.float32),
            jax.ShapeDtypeStruct((B * N,), jnp.int32),
        ),
        mesh=plsc.VectorSubcoreMesh(core_axis_name="c", subcore_axis_name="s"),
        compiler_params=pltpu.CompilerParams(needs_layout_passes=False),
        scratch_types=(
            [pltpu.VMEM((CHUNK,), jnp.float32) for _ in range(9)]
            + [pltpu.VMEM((T + 16,), jnp.float32) for _ in range(5)]
            + [pltpu.VMEM((T,), jnp.int32)]
            + [pltpu.VMEM((CHUNK,), jnp.float32),
               pltpu.VMEM((CHUNK,), jnp.int32),
               pltpu.VMEM((T * 16,), jnp.float32),
               pltpu.VMEM((T * 16,), jnp.int32),
               pltpu.VMEM((T,), jnp.float32),
               pltpu.VMEM((T,), jnp.int32),
               pltpu.VMEM((4 * T,), jnp.float32),
               pltpu.VMEM((4 * T,), jnp.int32),
               pltpu.VMEM((T + 16,), jnp.int32),
               pltpu.VMEM((CHUNK,), jnp.float32),
               pltpu.VMEM((CHUNK,), jnp.float32),
               pltpu.VMEM((CHUNK,), jnp.float32),
               pltpu.VMEM((CHUNK,), jnp.float32),
               pltpu.VMEM((CHUNK,), jnp.int32),
               pltpu.VMEM_SHARED((16 * T,), jnp.float32),
               pltpu.VMEM_SHARED((16 * T,), jnp.int32)]
        ),
    )(_body)
    loc_flat, lab_flat = k(pcx, pcy, pw, ph, b1, b2, b3, b4, labels)
    loc = loc_flat.reshape(B, 4, N).transpose(0, 2, 1)
    return loc, lab_flat.reshape(B, N)


# speedup vs baseline: 17.4861x; 1.2531x over previous
import functools

import jax
import jax.numpy as jnp
from jax import lax
from jax.experimental import pallas as pl
from jax.experimental.pallas import tpu as pltpu
from jax.experimental.pallas import tpu_sc as plsc

B, T, N = 8, 64, 20000
CHUNK = 5120
NJ = CHUNK // 16
GSTRIDE = 4960
LN2 = 0.6931472
SQRT2 = 1.4142135


def _log(x):
    bits = plsc.bitcast(x, jnp.int32)
    e = (bits >> 23) - 127
    mbits = (bits & 0x007FFFFF) | 0x3F800000
    m = plsc.bitcast(mbits, jnp.float32)
    big = m > SQRT2
    m = jnp.where(big, m * 0.5, m)
    e = jnp.where(big, e + 1, e)
    s = (m - 1.0) / (m + 1.0)
    s2 = s * s
    p = s * (2.0 + s2 * (0.66666667 + s2 * (0.4 + s2 * 0.2857143)))
    return e.astype(jnp.float32) * LN2 + p


def _body(pcx_hbm, pcy_hbm, pw_hbm, ph_hbm, bx1_hbm, by1_hbm, bx2_hbm, by2_hbm,
          labels_hbm, loc_hbm, lab_hbm,
          p_cx, p_cy, p_w, p_h, p_x1, p_y1, p_x2, p_y2, p_area,
          bx1, by1, bx2, by2, barea, blab,
          mval, midx, tbv, tbj, tred_v, tred_g, mg_v, mg_g, bp,
          lgx, lgy, lgw, lgh, lab_out, sh_v, sh_g):
    cid = lax.axis_index("c")
    sid = lax.axis_index("s")
    b = cid * 4 + sid // 4
    g = sid % 4
    base = g * GSTRIDE
    iota = lax.iota(jnp.int32, 16)

    pltpu.sync_copy(pcx_hbm.at[pl.ds(base, CHUNK)], p_cx)
    pltpu.sync_copy(pcy_hbm.at[pl.ds(base, CHUNK)], p_cy)
    pltpu.sync_copy(pw_hbm.at[pl.ds(base, CHUNK)], p_w)
    pltpu.sync_copy(ph_hbm.at[pl.ds(base, CHUNK)], p_h)
    bsl = pl.ds(b * T, T)
    pltpu.sync_copy(bx1_hbm.at[bsl], bx1.at[pl.ds(0, T)])
    pltpu.sync_copy(by1_hbm.at[bsl], by1.at[pl.ds(0, T)])
    pltpu.sync_copy(bx2_hbm.at[bsl], bx2.at[pl.ds(0, T)])
    pltpu.sync_copy(by2_hbm.at[bsl], by2.at[pl.ds(0, T)])
    pltpu.sync_copy(labels_hbm.at[bsl], blab)

    @plsc.parallel_loop(0, NJ, unroll=4)
    def _derive(j):
        sl = pl.ds(j * 16, 16)
        cx, cy, w, h = p_cx[sl], p_cy[sl], p_w[sl], p_h[sl]
        x1 = cx - w / 2.0
        y1 = cy - h / 2.0
        x2 = cx + w / 2.0
        y2 = cy + h / 2.0
        p_x1[sl] = x1
        p_y1[sl] = y1
        p_x2[sl] = x2
        p_y2[sl] = y2
        p_area[sl] = (x2 - x1) * (y2 - y1)

    def tinit(k, _):
        sl = pl.ds(k * 16, 16)
        x1, y1, x2, y2 = bx1[sl], by1[sl], bx2[sl], by2[sl]
        barea[sl] = (x2 - x1) * (y2 - y1)
        return 0

    lax.fori_loop(0, 4, tinit, 0)

    neg1 = jnp.full((16,), -1.0, jnp.float32)
    zero_i = jnp.full((16,), 0, jnp.int32)

    def tbinit(k, _):
        sl = pl.ds(k * 16, 16)
        tbv[sl] = neg1
        tbj[sl] = zero_i
        return 0

    lax.fori_loop(0, T, tbinit, 0)

    @plsc.parallel_loop(0, NJ, unroll=4)
    def _minit(j):
        sl = pl.ds(j * 16, 16)
        mval[sl] = neg1
        midx[sl] = zero_i

    def tloop(tp, _):
        t0 = tp * 2
        s0 = pl.ds(t0, 16)
        s1 = pl.ds(t0 + 1, 16)
        a0x1 = jnp.full((16,), bx1[s0][0], jnp.float32)
        a0y1 = jnp.full((16,), by1[s0][0], jnp.float32)
        a0x2 = jnp.full((16,), bx2[s0][0], jnp.float32)
        a0y2 = jnp.full((16,), by2[s0][0], jnp.float32)
        a0ar = jnp.full((16,), barea[s0][0], jnp.float32)
        a1x1 = jnp.full((16,), bx1[s1][0], jnp.float32)
        a1y1 = jnp.full((16,), by1[s1][0], jnp.float32)
        a1x2 = jnp.full((16,), bx2[s1][0], jnp.float32)
        a1y2 = jnp.full((16,), by2[s1][0], jnp.float32)
        a1ar = jnp.full((16,), barea[s1][0], jnp.float32)
        tvec0 = jnp.full((16,), t0, jnp.int32)
        tvec1 = jnp.full((16,), t0 + 1, jnp.int32)

        @plsc.parallel_loop(0, NJ, unroll=4,
                            carry=(neg1, zero_i, neg1, zero_i))
        def jloop(j, carry):
            tv0, tg0, tv1, tg1 = carry
            sl = pl.ds(j * 16, 16)
            px1, py1, px2, py2, pa = p_x1[sl], p_y1[sl], p_x2[sl], p_y2[sl], p_area[sl]
            wx0 = jnp.minimum(px2, a0x2) - jnp.maximum(px1, a0x1)
            wy0 = jnp.minimum(py2, a0y2) - jnp.maximum(py1, a0y1)
            inter0 = jnp.maximum(wx0, 0.0) * jnp.maximum(wy0, 0.0)
            iou0 = inter0 / (((a0ar + pa) - inter0) + 1e-12)
            wx1 = jnp.minimum(px2, a1x2) - jnp.maximum(px1, a1x1)
            wy1 = jnp.minimum(py2, a1y2) - jnp.maximum(py1, a1y1)
            inter1 = jnp.maximum(wx1, 0.0) * jnp.maximum(wy1, 0.0)
            iou1 = inter1 / (((a1ar + pa) - inter1) + 1e-12)
            bv = mval[sl]
            bi = midx[sl]
            c0 = iou0 > bv
            bv = jnp.where(c0, iou0, bv)
            bi = jnp.where(c0, tvec0, bi)
            c1 = iou1 > bv
            mval[sl] = jnp.where(c1, iou1, bv)
            midx[sl] = jnp.where(c1, tvec1, bi)
            jv = jnp.full((16,), j, jnp.int32)
            take0 = iou0 > tv0
            tv0 = jnp.where(take0, iou0, tv0)
            tg0 = jnp.where(take0, jv, tg0)
            take1 = iou1 > tv1
            tv1 = jnp.where(take1, iou1, tv1)
            tg1 = jnp.where(take1, jv, tg1)
            return tv0, tg0, tv1, tg1

        tv0, tg0, tv1, tg1 = jloop
        tbv[pl.ds(t0 * 16, 16)] = tv0
        tbj[pl.ds(t0 * 16, 16)] = tg0
        tbv[pl.ds((t0 + 1) * 16, 16)] = tv1
        tbj[pl.ds((t0 + 1) * 16, 16)] = tg1
        return 0

    lax.fori_loop(0, T // 2, tloop, 0)

    for tgrp in range(4):
        cur_v = neg1
        cur_g = zero_i
        tvec = jnp.full((16,), tgrp * 16, jnp.int32) + iota
        for l in range(16):
            idx = tvec * 16 + l
            v_l = plsc.load_gather(tbv, [idx])
            j_l = plsc.load_gather(tbj, [idx])
            g_l = base + (j_l * 16 + l)
            take = (v_l > cur_v) | ((v_l == cur_v) & (g_l < cur_g))
            cur_v = jnp.where(take, v_l, cur_v)
            cur_g = jnp.where(take, g_l, cur_g)
        osl = pl.ds(tgrp * 16, 16)
        tred_v[osl] = cur_v
        tred_g[osl] = cur_g

    pltpu.sync_copy(tred_v, sh_v.at[pl.ds(sid * T, T)])
    pltpu.sync_copy(tred_g, sh_g.at[pl.ds(sid * T, T)])
    plsc.subcore_barrier()
    grp0 = (sid // 4) * 4
    pltpu.sync_copy(sh_v.at[pl.ds(grp0 * T, 4 * T)], mg_v)
    pltpu.sync_copy(sh_g.at[pl.ds(grp0 * T, 4 * T)], mg_g)
    for tt in range(4):
        cur_v = neg1
        cur_g = zero_i
        for gg in range(4):
            sl = pl.ds(gg * T + tt * 16, 16)
            v = mg_v[sl]
            gi = mg_g[sl]
            take = (v > cur_v) | ((v == cur_v) & (gi < cur_g))
            cur_v = jnp.where(take, v, cur_v)
            cur_g = jnp.where(take, gi, cur_g)
        bp[pl.ds(tt * 16, 16)] = cur_g

    lane0 = iota == 0

    def force(t, _):
        lp = bp[pl.ds(t, 16)][0] - base

        @pl.when((lp >= 0) & (lp < CHUNK))
        def _():
            li = jnp.full((16,), lp, jnp.int32)
            plsc.store_scatter(midx, [li], jnp.full((16,), t, jnp.int32), mask=lane0)
            plsc.store_scatter(mval, [li], jnp.full((16,), 2.0, jnp.float32), mask=lane0)

        return 0

    lax.fori_loop(0, T, force, 0)

    @plsc.parallel_loop(0, NJ, unroll=4)
    def _encode(j):
        sl = pl.ds(j * 16, 16)
        m = midx[sl]
        v = mval[sl]
        lab = plsc.load_gather(blab, [m])
        lab_out[sl] = jnp.where(v < 0.5, jnp.full((16,), 0, jnp.int32), lab)
        m_x1 = plsc.load_gather(bx1, [m])
        m_y1 = plsc.load_gather(by1, [m])
        m_x2 = plsc.load_gather(bx2, [m])
        m_y2 = plsc.load_gather(by2, [m])
        cx, cy, w, h = p_cx[sl], p_cy[sl], p_w[sl], p_h[sl]
        g_cx = ((m_x1 + m_x2) / 2.0 - cx) / (0.1 * w)
        g_cy = ((m_y1 + m_y2) / 2.0 - cy) / (0.1 * h)
        g_w = _log(jnp.maximum((m_x2 - m_x1) / w, 1e-8)) / 0.2
        g_h = _log(jnp.maximum((m_y2 - m_y1) / h, 1e-8)) / 0.2
        lgx[sl] = g_cx
        lgy[sl] = g_cy
        lgw[sl] = g_w
        lgh[sl] = g_h

    pltpu.sync_copy(lgx, loc_hbm.at[pl.ds((b * 4 + 0) * N + base, CHUNK)])
    pltpu.sync_copy(lgy, loc_hbm.at[pl.ds((b * 4 + 1) * N + base, CHUNK)])
    pltpu.sync_copy(lgw, loc_hbm.at[pl.ds((b * 4 + 2) * N + base, CHUNK)])
    pltpu.sync_copy(lgh, loc_hbm.at[pl.ds((b * 4 + 3) * N + base, CHUNK)])
    pltpu.sync_copy(lab_out, lab_hbm.at[pl.ds(b * N + base, CHUNK)])


@jax.jit
def kernel(priors_xywha, gt_boxes, gt_labels):
    pcx, pcy, pw, ph = [jnp.reshape(priors_xywha[:, i], (N,)) for i in range(4)]
    b1, b2, b3, b4 = [jnp.reshape(gt_boxes[:, :, i], (B * T,)) for i in range(4)]
    labels = jnp.reshape(gt_labels.astype(jnp.int32), (B * T,))

    k = functools.partial(
        pl.kernel,
        out_type=(
            jax.ShapeDtypeStruct((B * N * 4,), jnp.float32),
            jax.ShapeDtypeStruct((B * N,), jnp.int32),
        ),
        mesh=plsc.VectorSubcoreMesh(core_axis_name="c", subcore_axis_name="s"),
        compiler_params=pltpu.CompilerParams(needs_layout_passes=False),
        scratch_types=(
            [pltpu.VMEM((CHUNK,), jnp.float32) for _ in range(9)]
            + [pltpu.VMEM((T + 16,), jnp.float32) for _ in range(5)]
            + [pltpu.VMEM((T,), jnp.int32)]
            + [pltpu.VMEM((CHUNK,), jnp.float32),
               pltpu.VMEM((CHUNK,), jnp.int32),
               pltpu.VMEM((T * 16,), jnp.float32),
               pltpu.VMEM((T * 16,), jnp.int32),
               pltpu.VMEM((T,), jnp.float32),
               pltpu.VMEM((T,), jnp.int32),
               pltpu.VMEM((4 * T,), jnp.float32),
               pltpu.VMEM((4 * T,), jnp.int32),
               pltpu.VMEM((T + 16,), jnp.int32),
               pltpu.VMEM((CHUNK,), jnp.float32),
               pltpu.VMEM((CHUNK,), jnp.float32),
               pltpu.VMEM((CHUNK,), jnp.float32),
               pltpu.VMEM((CHUNK,), jnp.float32),
               pltpu.VMEM((CHUNK,), jnp.int32),
               pltpu.VMEM_SHARED((16 * T,), jnp.float32),
               pltpu.VMEM_SHARED((16 * T,), jnp.int32)]
        ),
    )(_body)
    loc_flat, lab_flat = k(pcx, pcy, pw, ph, b1, b2, b3, b4, labels)
    loc = loc_flat.reshape(B, 4, N).transpose(0, 2, 1)
    return loc, lab_flat.reshape(B, N)
